# baseline (speedup 1.0000x reference)
import functools

import jax
import jax.numpy as jnp
from jax import lax
from jax.experimental import pallas as pl
from jax.experimental.pallas import tpu as pltpu
from jax.experimental.pallas import tpu_sc as plsc

_ROWS = 8
_IPAD = 64
_HPAD = 56


def _emb_lookup(idx, table, hist):
    nb = idx.shape[0]
    dim = table.shape[1]
    info = plsc.get_sparse_core_info()
    nw = info.num_cores * info.num_subcores
    rows_per_w = nb // nw
    n_chunks = rows_per_w // _ROWS
    mesh = plsc.VectorSubcoreMesh(core_axis_name="c", subcore_axis_name="s")

    @functools.partial(
        pl.kernel,
        mesh=mesh,
        compiler_params=pltpu.CompilerParams(use_tc_tiling_on_sc=False),
        out_type=jax.ShapeDtypeStruct((nb, hist, dim), jnp.float32),
        scratch_types=[
            pltpu.VMEM((rows_per_w, _IPAD), jnp.int32),
            pltpu.VMEM((_ROWS, _HPAD, dim), jnp.float32),
            pltpu.VMEM((_ROWS, _HPAD, dim), jnp.float32),
            pltpu.SemaphoreType.DMA,
            pltpu.SemaphoreType.DMA,
            pltpu.SemaphoreType.DMA,
            pltpu.SemaphoreType.DMA,
        ],
    )
    def emb(idx_hbm, tab_hbm, out_hbm, idx_all, rows0, rows1,
            gsem0, gsem1, osem0, osem1):
        wid = lax.axis_index("s") * info.num_cores + lax.axis_index("c")
        base = wid * rows_per_w
        rows = (rows0, rows1)
        gsem = (gsem0, gsem1)
        osem = (osem0, osem1)

        pltpu.sync_copy(
            idx_hbm.at[pl.ds(pl.multiple_of(base, 8), rows_per_w)], idx_all)

        def fire(chunk, slot):
            return [
                pltpu.async_copy(
                    tab_hbm.at[idx_all.at[chunk * _ROWS + j, pl.ds(0, _HPAD)]],
                    rows[slot].at[j],
                    gsem[slot],
                )
                for j in range(_ROWS)
            ]

        def start_out(chunk, slot):
            return pltpu.async_copy(
                rows[slot].at[:, pl.ds(0, hist)],
                out_hbm.at[pl.ds(base + chunk * _ROWS, _ROWS)],
                osem[slot],
            )

        def drain_out(slot):
            pltpu.make_async_copy(
                rows[slot].at[:, pl.ds(0, hist)],
                out_hbm.at[pl.ds(base, _ROWS)],
                osem[slot],
            ).wait()

        def body(k, carry):
            a = 2 * k
            b = a + 1

            @pl.when(k > 0)
            def _():
                drain_out(0)

            ga = fire(a, 0)

            @pl.when(k > 0)
            def _():
                drain_out(1)

            gb = fire(b, 1)
            for c in ga:
                c.wait()
            start_out(a, 0)
            for c in gb:
                c.wait()
            start_out(b, 1)
            return carry

        lax.fori_loop(0, n_chunks // 2, body, 0)
        drain_out(0)
        drain_out(1)

    return emb(idx, table)


def kernel(input_ids, table):
    hist = input_ids.shape[1]
    idx_pad = jnp.pad(input_ids.astype(jnp.int32),
                      ((0, 0), (0, _IPAD - hist)))
    return _emb_lookup(idx_pad, table, hist)


# speedup vs baseline: 2.8624x; 2.8624x over previous
import functools

import jax
import jax.numpy as jnp
from jax import lax
from jax.experimental import pallas as pl
from jax.experimental.pallas import tpu as pltpu
from jax.experimental.pallas import tpu_sc as plsc

_LANES = 128
_CHUNK = 512
_KSUB = _CHUNK // _LANES


def _emb_lookup(idx2d, table2, n_rows):
    info = plsc.get_sparse_core_info()
    nw = info.num_cores * info.num_subcores
    b_per_w = n_rows // nw
    n_chunks = b_per_w // _CHUNK
    idx_rows = b_per_w // _LANES
    dim = table2.shape[1]
    mesh = plsc.VectorSubcoreMesh(core_axis_name="c", subcore_axis_name="s")

    @functools.partial(
        pl.kernel,
        mesh=mesh,
        compiler_params=pltpu.CompilerParams(use_tc_tiling_on_sc=False),
        out_type=jax.ShapeDtypeStruct((n_rows, dim), jnp.float32),
        scratch_types=[
            pltpu.VMEM((idx_rows, _LANES), jnp.int32),
            pltpu.VMEM((_CHUNK, dim), jnp.float32),
            pltpu.VMEM((_CHUNK, dim), jnp.float32),
            pltpu.SemaphoreType.DMA,
            pltpu.SemaphoreType.DMA,
            pltpu.SemaphoreType.DMA,
            pltpu.SemaphoreType.DMA,
        ],
    )
    def emb(idx_hbm, tab_hbm, out_hbm, idx_all, rows0, rows1,
            gsem0, gsem1, osem0, osem1):
        wid = lax.axis_index("s") * info.num_cores + lax.axis_index("c")
        base = wid * b_per_w
        rows = (rows0, rows1)
        gsem = (gsem0, gsem1)
        osem = (osem0, osem1)

        pltpu.sync_copy(
            idx_hbm.at[pl.ds(pl.multiple_of(base // _LANES, 8), idx_rows)],
            idx_all)

        def fire(chunk, slot):
            return [
                pltpu.async_copy(
                    tab_hbm.at[idx_all.at[chunk * _KSUB + j]],
                    rows[slot].at[pl.ds(j * _LANES, _LANES)],
                    gsem[slot],
                )
                for j in range(_KSUB)
            ]

        def start_out(chunk, slot):
            return pltpu.async_copy(
                rows[slot],
                out_hbm.at[pl.ds(base + chunk * _CHUNK, _CHUNK)],
                osem[slot],
            )

        def drain_out(slot):
            pltpu.make_async_copy(
                rows[slot],
                out_hbm.at[pl.ds(0, _CHUNK)],
                osem[slot],
            ).wait()

        def body(k, carry):
            a = 2 * k
            b = a + 1

            @pl.when(k > 0)
            def _():
                drain_out(0)

            ga = fire(a, 0)

            @pl.when(k > 0)
            def _():
                drain_out(1)

            gb = fire(b, 1)
            for c in ga:
                c.wait()
            start_out(a, 0)
            for c in gb:
                c.wait()
            start_out(b, 1)
            return carry

        lax.fori_loop(0, n_chunks // 2, body, 0)
        drain_out(0)
        drain_out(1)

    return emb(idx2d, table2)


def kernel(input_ids, table):
    b, h = input_ids.shape
    n = b * h
    v, dim = table.shape
    table2 = jnp.pad(table, ((0, 0), (0, 128 - dim))).reshape(2 * v, dim)
    idx2d = (input_ids.reshape(n // _LANES, _LANES).astype(jnp.int32)) * 2
    out = _emb_lookup(idx2d, table2, n)
    return out.reshape(b, h, dim)


# speedup vs baseline: 2.8701x; 1.0027x over previous
import functools

import jax
import jax.numpy as jnp
from jax import lax
from jax.experimental import pallas as pl
from jax.experimental.pallas import tpu as pltpu
from jax.experimental.pallas import tpu_sc as plsc

_LANES = 128
_CHUNK = 256
_KSUB = _CHUNK // _LANES


def _emb_lookup(idx2d, table2, n_rows):
    info = plsc.get_sparse_core_info()
    nw = info.num_cores * info.num_subcores
    b_per_w = n_rows // nw
    n_chunks = b_per_w // _CHUNK
    idx_rows = b_per_w // _LANES
    dim = table2.shape[1]
    mesh = plsc.VectorSubcoreMesh(core_axis_name="c", subcore_axis_name="s")

    @functools.partial(
        pl.kernel,
        mesh=mesh,
        compiler_params=pltpu.CompilerParams(use_tc_tiling_on_sc=False),
        out_type=jax.ShapeDtypeStruct((n_rows, dim), jnp.float32),
        scratch_types=[
            pltpu.VMEM((idx_rows, _LANES), jnp.int32),
            pltpu.VMEM((_CHUNK, dim), jnp.float32),
            pltpu.VMEM((_CHUNK, dim), jnp.float32),
            pltpu.VMEM((_CHUNK, dim), jnp.float32),
            pltpu.VMEM((_CHUNK, dim), jnp.float32),
            pltpu.SemaphoreType.DMA,
            pltpu.SemaphoreType.DMA,
            pltpu.SemaphoreType.DMA,
            pltpu.SemaphoreType.DMA,
            pltpu.SemaphoreType.DMA,
            pltpu.SemaphoreType.DMA,
            pltpu.SemaphoreType.DMA,
            pltpu.SemaphoreType.DMA,
        ],
    )
    def emb(idx_hbm, tab_hbm, out_hbm, idx_all, rows0, rows1, rows2, rows3,
            gsem0, gsem1, gsem2, gsem3, osem0, osem1, osem2, osem3):
        wid = lax.axis_index("s") * info.num_cores + lax.axis_index("c")
        base = wid * b_per_w
        rows = (rows0, rows1, rows2, rows3)
        gsem = (gsem0, gsem1, gsem2, gsem3)
        osem = (osem0, osem1, osem2, osem3)

        pltpu.sync_copy(
            idx_hbm.at[pl.ds(pl.multiple_of(base // _LANES, 8), idx_rows)],
            idx_all)

        def fire(chunk, slot):
            return [
                pltpu.async_copy(
                    tab_hbm.at[idx_all.at[chunk * _KSUB + j]],
                    rows[slot].at[pl.ds(j * _LANES, _LANES)],
                    gsem[slot],
                )
                for j in range(_KSUB)
            ]

        def start_out(chunk, slot):
            return pltpu.async_copy(
                rows[slot],
                out_hbm.at[pl.ds(base + chunk * _CHUNK, _CHUNK)],
                osem[slot],
            )

        def drain_out(slot):
            pltpu.make_async_copy(
                rows[slot],
                out_hbm.at[pl.ds(0, _CHUNK)],
                osem[slot],
            ).wait()

        nbuf = len(rows)

        def body(k, carry):
            gs = []
            for s in range(nbuf):
                @pl.when(k > 0)
                def _(s=s):
                    drain_out(s)

                gs.append(fire(nbuf * k + s, s))
            for s in range(nbuf):
                for c in gs[s]:
                    c.wait()
                start_out(nbuf * k + s, s)
            return carry

        lax.fori_loop(0, n_chunks // nbuf, body, 0)
        for s in range(nbuf):
            drain_out(s)

    return emb(idx2d, table2)


def kernel(input_ids, table):
    b, h = input_ids.shape
    n = b * h
    v, dim = table.shape
    table2 = jnp.pad(table, ((0, 0), (0, 128 - dim))).reshape(2 * v, dim)
    idx2d = (input_ids.reshape(n // _LANES, _LANES).astype(jnp.int32)) * 2
    out = _emb_lookup(idx2d, table2, n)
    return out.reshape(b, h, dim)
